# baseline (speedup 1.0000x reference)
import jax
import jax.numpy as jnp
from jax.experimental import pallas as pl
from jax.experimental.pallas import tpu as pltpu

_NUM_CLASSES = 8192
_Z_DIM = 256
_SIZES = (1024, 1024, 1024, 1024, 1024, 1024, 1024, 1024)
_OFFS = tuple(sum(_SIZES[:i]) for i in range(len(_SIZES)))
_K = len(_SIZES)


def _in_copy(a_hbm, buf, in_sems, i):
    return pltpu.make_async_copy(
        a_hbm.at[pl.ds(_OFFS[i], _SIZES[i]), :],
        buf.at[pl.ds(_OFFS[i], _SIZES[i]), :],
        in_sems.at[i],
    )


def _out_copy(o_hbm, buf, out_sems, i):
    return pltpu.make_async_copy(
        buf.at[pl.ds(_OFFS[i], _SIZES[i]), :],
        o_hbm.at[pl.ds(_OFFS[i], _SIZES[i]), :],
        out_sems.at[i],
    )


def _copy_body(a_hbm, o_hbm, buf, in_sems, out_sems):
    _in_copy(a_hbm, buf, in_sems, 0).start()
    for i in range(_K):
        if i + 1 < _K:
            _in_copy(a_hbm, buf, in_sems, i + 1).start()
        _in_copy(a_hbm, buf, in_sems, i).wait()
        _out_copy(o_hbm, buf, out_sems, i).start()
    for i in range(_K):
        _out_copy(o_hbm, buf, out_sems, i).wait()


def kernel(_, anchor):
    return pl.pallas_call(
        _copy_body,
        in_specs=[pl.BlockSpec(memory_space=pl.ANY)],
        out_specs=pl.BlockSpec(memory_space=pl.ANY),
        out_shape=jax.ShapeDtypeStruct((_NUM_CLASSES, _Z_DIM), jnp.float32),
        scratch_shapes=[
            pltpu.VMEM((_NUM_CLASSES, _Z_DIM), jnp.float32),
            pltpu.SemaphoreType.DMA((_K,)),
            pltpu.SemaphoreType.DMA((_K,)),
        ],
    )(anchor)


# speedup vs baseline: 1.2934x; 1.2934x over previous
import jax
import jax.numpy as jnp
from jax.experimental import pallas as pl
from jax.experimental.pallas import tpu as pltpu

_NUM_CLASSES = 8192
_Z_DIM = 256
_SIZES = (256, 256, 512, 768, 1024, 1280, 1792, 2304)
_OFFS = tuple(sum(_SIZES[:i]) for i in range(len(_SIZES)))
_K = len(_SIZES)


def _in_copy(a_hbm, buf, in_sems, i):
    return pltpu.make_async_copy(
        a_hbm.at[pl.ds(_OFFS[i], _SIZES[i]), :],
        buf.at[pl.ds(_OFFS[i], _SIZES[i]), :],
        in_sems.at[i],
    )


def _out_copy(o_hbm, buf, out_sems, i):
    return pltpu.make_async_copy(
        buf.at[pl.ds(_OFFS[i], _SIZES[i]), :],
        o_hbm.at[pl.ds(_OFFS[i], _SIZES[i]), :],
        out_sems.at[i],
    )


def _copy_body(a_hbm, o_hbm, buf, in_sems, out_sems):
    for i in range(_K):
        _in_copy(a_hbm, buf, in_sems, i).start()
    for i in range(_K):
        _in_copy(a_hbm, buf, in_sems, i).wait()
        _out_copy(o_hbm, buf, out_sems, i).start()
    for i in range(_K):
        _out_copy(o_hbm, buf, out_sems, i).wait()


def kernel(_, anchor):
    return pl.pallas_call(
        _copy_body,
        in_specs=[pl.BlockSpec(memory_space=pl.ANY)],
        out_specs=pl.BlockSpec(memory_space=pl.ANY),
        out_shape=jax.ShapeDtypeStruct((_NUM_CLASSES, _Z_DIM), jnp.float32),
        scratch_shapes=[
            pltpu.VMEM((_NUM_CLASSES, _Z_DIM), jnp.float32),
            pltpu.SemaphoreType.DMA((_K,)),
            pltpu.SemaphoreType.DMA((_K,)),
        ],
    )(anchor)
